# baseline (speedup 1.0000x reference)
import jax
import jax.numpy as jnp
import numpy as np
from jax.experimental import pallas as pl
from jax.experimental.pallas import tpu as pltpu

IMAGE_W = 640
IMAGE_H = 480
FOV = 55.0
FOCAL = np.float32(IMAGE_W / (2.0 * np.tan(np.radians(FOV / 2.0))))
NG = 2048
TH = 16
NROWS = 16


def _project_kernel(camx_r, camy_r, camz_r, s0, s1, s2, c0, c1, c2, op, cp, out):
    camx = camx_r[...]
    camy = camy_r[...]
    camz = camz_r[...]
    depth = jnp.maximum(-camz, 0.1)
    sx = FOCAL * camx / depth + IMAGE_W / 2.0
    sy = FOCAL * camy / depth + IMAGE_H / 2.0
    valid = (sx >= 0) & (sx < IMAGE_W) & (sy >= 0) & (sy < IMAGE_H)
    sm = (s0[...] + s1[...] + s2[...]) / 3.0
    ss = jnp.clip(sm * FOCAL / depth, 1.0, 20.0)
    radf = jnp.floor(ss * 3.0)
    xi = jnp.clip(jnp.floor(sx), 0.0, IMAGE_W - 1.0)
    yi = jnp.clip(jnp.floor(sy), 0.0, IMAGE_H - 1.0)
    big = jnp.float32(1e9)
    lox = jnp.where(valid, xi - radf, big)
    hix = jnp.where(valid, xi + radf, -big)
    loy = jnp.where(valid, yi - radf, big)
    hiy = jnp.where(valid, yi + radf, -big)
    opv = jnp.where(valid, op[...], 0.0)
    ninv = -0.5 / (ss * ss)

    def put(r, v):
        out[pl.ds(r, 1), :] = v.reshape(1, NG)

    put(0, sx)
    put(1, sy)
    put(2, ninv)
    put(3, opv)
    put(4, lox)
    put(5, hix)
    put(6, loy)
    put(7, hiy)
    put(8, c0[...])
    put(9, c1[...])
    put(10, c2[...])
    put(11, jnp.zeros_like(sx))
    put(12, jnp.zeros_like(sx))
    put(13, jnp.zeros_like(sx))
    put(14, jnp.zeros_like(sx))
    put(15, jnp.zeros_like(sx))


def _compact_kernel(params, cparams, cnt):
    def body(k, c):
        pred = params[7, k] >= params[6, k]

        @pl.when(pred)
        def _():
            for r in range(11):
                cparams[r, c] = params[r, k]

        return c + jnp.where(pred, 1, 0)

    n = jax.lax.fori_loop(0, NG, body, jnp.int32(0))
    cnt[0, 0] = n


NPAN = IMAGE_W // 128


def _raster_kernel(cparams, cnt, out, tbuf):
    y0 = pl.program_id(0) * TH
    y0f = jnp.float32(0) + y0
    y1f = y0f + (TH - 1)
    pxl = jax.lax.broadcasted_iota(jnp.int32, (1, 128), 1).astype(jnp.float32)
    pyf = y0f + jax.lax.broadcasted_iota(jnp.int32, (TH, 1), 0).astype(jnp.float32)
    out[...] = jnp.zeros((NPAN, 3, TH, 128), jnp.float32)
    tbuf[...] = jnp.ones((NPAN, TH, 128), jnp.float32)
    n = cnt[0, 0]

    def body(i, _):
        k = n - 1 - i
        loy = cparams[6, k]
        hiy = cparams[7, k]

        @pl.when((hiy >= y0f) & (loy <= y1f))
        def _():
            sx = cparams[0, k]
            sy = cparams[1, k]
            ninv = cparams[2, k]
            opv = cparams[3, k]
            lox = cparams[4, k]
            hix = cparams[5, k]
            c0 = cparams[8, k]
            c1 = cparams[9, k]
            c2 = cparams[10, k]
            ddy = pyf - sy
            wy = jnp.where((pyf >= loy) & (pyf <= hiy),
                           jnp.exp(ninv * (ddy * ddy)), 0.0)
            p_lo = jnp.clip(jnp.floor(lox * (1.0 / 128.0)), 0.0,
                            NPAN - 1.0).astype(jnp.int32)
            p_hi = jnp.clip(jnp.floor(hix * (1.0 / 128.0)), 0.0,
                            NPAN - 1.0).astype(jnp.int32)

            def pbody(p, _):
                pxf = pxl + jnp.float32(128.0) * p.astype(jnp.float32)
                ddx = pxf - sx
                wx = jnp.where((pxf >= lox) & (pxf <= hix),
                               opv * jnp.exp(ninv * (ddx * ddx)), 0.0)
                am = (wy * wx) * tbuf[p]
                tbuf[p] = tbuf[p] - am
                out[p, 0] = out[p, 0] + am * c0
                out[p, 1] = out[p, 1] + am * c1
                out[p, 2] = out[p, 2] + am * c2
                return 0

            jax.lax.fori_loop(p_lo, p_hi + 1, pbody, 0)

        return 0

    jax.lax.fori_loop(0, n, body, 0)


def kernel(positions, scales, rotations, colors, opacities, camera_pose):
    del rotations
    R = camera_pose[:3, :3]
    t = camera_pose[:3, 3]
    cam = (positions - t) @ R.T
    plane = lambda a: a.reshape(8, 256)
    args = [plane(cam[:, 0]), plane(cam[:, 1]), plane(cam[:, 2]),
            plane(scales[:, 0]), plane(scales[:, 1]), plane(scales[:, 2]),
            plane(colors[:, 0]), plane(colors[:, 1]), plane(colors[:, 2]),
            plane(opacities), camera_pose]
    vspec = pl.BlockSpec((8, 256), lambda: (0, 0))
    params = pl.pallas_call(
        _project_kernel,
        out_shape=jax.ShapeDtypeStruct((NROWS, NG), jnp.float32),
        in_specs=[vspec] * 10 + [pl.BlockSpec(memory_space=pltpu.SMEM)],
        out_specs=pl.BlockSpec((NROWS, NG), lambda: (0, 0)),
    )(*args)

    smem_spec = pl.BlockSpec(memory_space=pltpu.SMEM)
    cparams, cnt = pl.pallas_call(
        _compact_kernel,
        out_shape=(jax.ShapeDtypeStruct((NROWS, NG), jnp.float32),
                   jax.ShapeDtypeStruct((1, 1), jnp.int32)),
        in_specs=[smem_spec],
        out_specs=(smem_spec, smem_spec),
    )(params)

    imgp = pl.pallas_call(
        _raster_kernel,
        grid=(IMAGE_H // TH,),
        out_shape=jax.ShapeDtypeStruct((NPAN, 3, IMAGE_H, 128), jnp.float32),
        in_specs=[pl.BlockSpec((NROWS, NG), lambda i: (0, 0),
                               memory_space=pltpu.SMEM),
                  pl.BlockSpec((1, 1), lambda i: (0, 0),
                               memory_space=pltpu.SMEM)],
        out_specs=pl.BlockSpec((NPAN, 3, TH, 128), lambda i: (0, 0, i, 0)),
        scratch_shapes=[pltpu.VMEM((NPAN, TH, 128), jnp.float32)],
        compiler_params=pltpu.CompilerParams(
            dimension_semantics=("parallel",)),
    )(cparams, cnt)
    return imgp.transpose(1, 2, 0, 3).reshape(3, IMAGE_H, IMAGE_W)


# speedup vs baseline: 1.1360x; 1.1360x over previous
import jax
import jax.numpy as jnp
import numpy as np
from jax.experimental import pallas as pl
from jax.experimental.pallas import tpu as pltpu

IMAGE_W = 640
IMAGE_H = 480
FOV = 55.0
FOCAL = np.float32(IMAGE_W / (2.0 * np.tan(np.radians(FOV / 2.0))))
NG = 2048
TH = 16
NROWS = 16


def _project_kernel(camx_r, camy_r, camz_r, s0, s1, s2, c0, c1, c2, op, cp, out):
    camx = camx_r[...]
    camy = camy_r[...]
    camz = camz_r[...]
    depth = jnp.maximum(-camz, 0.1)
    sx = FOCAL * camx / depth + IMAGE_W / 2.0
    sy = FOCAL * camy / depth + IMAGE_H / 2.0
    valid = (sx >= 0) & (sx < IMAGE_W) & (sy >= 0) & (sy < IMAGE_H)
    sm = (s0[...] + s1[...] + s2[...]) / 3.0
    ss = jnp.clip(sm * FOCAL / depth, 1.0, 20.0)
    radf = jnp.floor(ss * 3.0)
    xi = jnp.clip(jnp.floor(sx), 0.0, IMAGE_W - 1.0)
    yi = jnp.clip(jnp.floor(sy), 0.0, IMAGE_H - 1.0)
    big = jnp.float32(1e9)
    lox = jnp.where(valid, xi - radf, big)
    hix = jnp.where(valid, xi + radf, -big)
    loy = jnp.where(valid, yi - radf, big)
    hiy = jnp.where(valid, yi + radf, -big)
    opv = jnp.where(valid, op[...], 0.0)
    ninv = -0.5 / (ss * ss)

    def put(r, v):
        out[pl.ds(r, 1), :] = v.reshape(1, NG)

    put(0, sx)
    put(1, sy)
    put(2, ninv)
    put(3, opv)
    put(4, lox)
    put(5, hix)
    put(6, loy)
    put(7, hiy)
    put(8, c0[...])
    put(9, c1[...])
    put(10, c2[...])
    put(11, jnp.zeros_like(sx))
    put(12, jnp.zeros_like(sx))
    put(13, jnp.zeros_like(sx))
    put(14, jnp.zeros_like(sx))
    put(15, jnp.zeros_like(sx))


def _raster_kernel(params, out, cparams, cnt, tbuf):
    @pl.when(pl.program_id(0) == 0)
    def _():
        def cbody(k, c):
            pred = params[7, k] >= params[6, k]

            @pl.when(pred)
            def _():
                for r in range(11):
                    cparams[r, c] = params[r, k]

            return c + jnp.where(pred, 1, 0)

        cnt[0] = jax.lax.fori_loop(0, NG, cbody, jnp.int32(0))

    y0 = pl.program_id(0) * TH
    y0f = jnp.float32(0) + y0
    y1f = y0f + (TH - 1)
    pxf = jax.lax.broadcasted_iota(jnp.int32, (1, IMAGE_W), 1).astype(jnp.float32)
    pyf = y0f + jax.lax.broadcasted_iota(jnp.int32, (TH, 1), 0).astype(jnp.float32)
    out[...] = jnp.zeros((3, TH, IMAGE_W), jnp.float32)
    tbuf[...] = jnp.ones((TH, IMAGE_W), jnp.float32)
    n = cnt[0]

    def body(i, _):
        k = n - 1 - i
        loy = cparams[6, k]
        hiy = cparams[7, k]

        @pl.when((hiy >= y0f) & (loy <= y1f))
        def _():
            sx = cparams[0, k]
            sy = cparams[1, k]
            ninv = cparams[2, k]
            opv = cparams[3, k]
            lox = cparams[4, k]
            hix = cparams[5, k]
            ddx = pxf - sx
            wx = jnp.where((pxf >= lox) & (pxf <= hix),
                           opv * jnp.exp(ninv * (ddx * ddx)), 0.0)
            ddy = pyf - sy
            wy = jnp.where((pyf >= loy) & (pyf <= hiy),
                           jnp.exp(ninv * (ddy * ddy)), 0.0)
            am = (wy * wx) * tbuf[...]
            tbuf[...] = tbuf[...] - am
            out[0] = out[0] + am * cparams[8, k]
            out[1] = out[1] + am * cparams[9, k]
            out[2] = out[2] + am * cparams[10, k]

        return 0

    jax.lax.fori_loop(0, n, body, 0)


def kernel(positions, scales, rotations, colors, opacities, camera_pose):
    del rotations
    R = camera_pose[:3, :3]
    t = camera_pose[:3, 3]
    cam = (positions - t) @ R.T
    plane = lambda a: a.reshape(8, 256)
    args = [plane(cam[:, 0]), plane(cam[:, 1]), plane(cam[:, 2]),
            plane(scales[:, 0]), plane(scales[:, 1]), plane(scales[:, 2]),
            plane(colors[:, 0]), plane(colors[:, 1]), plane(colors[:, 2]),
            plane(opacities), camera_pose]
    vspec = pl.BlockSpec((8, 256), lambda: (0, 0))
    params = pl.pallas_call(
        _project_kernel,
        out_shape=jax.ShapeDtypeStruct((NROWS, NG), jnp.float32),
        in_specs=[vspec] * 10 + [pl.BlockSpec(memory_space=pltpu.SMEM)],
        out_specs=pl.BlockSpec((NROWS, NG), lambda: (0, 0)),
    )(*args)

    img = pl.pallas_call(
        _raster_kernel,
        grid=(IMAGE_H // TH,),
        out_shape=jax.ShapeDtypeStruct((3, IMAGE_H, IMAGE_W), jnp.float32),
        in_specs=[pl.BlockSpec((NROWS, NG), lambda i: (0, 0),
                               memory_space=pltpu.SMEM)],
        out_specs=pl.BlockSpec((3, TH, IMAGE_W), lambda i: (0, i, 0)),
        scratch_shapes=[pltpu.SMEM((NROWS, NG), jnp.float32),
                        pltpu.SMEM((1,), jnp.int32),
                        pltpu.VMEM((TH, IMAGE_W), jnp.float32)],
    )(params)
    return img


# speedup vs baseline: 1.2967x; 1.1415x over previous
import jax
import jax.numpy as jnp
import numpy as np
from jax.experimental import pallas as pl
from jax.experimental.pallas import tpu as pltpu

IMAGE_W = 640
IMAGE_H = 480
FOV = 55.0
FOCAL = np.float32(IMAGE_W / (2.0 * np.tan(np.radians(FOV / 2.0))))
NG = 2048
TH = 32
NROWS = 11


def _project_kernel(gin, out):
    camx = gin[0]
    camy = gin[1]
    camz = gin[2]
    s0, s1, s2 = gin[3], gin[4], gin[5]
    c0, c1, c2 = gin[6], gin[7], gin[8]
    op = gin[9]
    depth = jnp.maximum(-camz, 0.1)
    sx = FOCAL * camx / depth + IMAGE_W / 2.0
    sy = FOCAL * camy / depth + IMAGE_H / 2.0
    valid = (sx >= 0) & (sx < IMAGE_W) & (sy >= 0) & (sy < IMAGE_H)
    sm = (s0 + s1 + s2) / 3.0
    ss = jnp.clip(sm * FOCAL / depth, 1.0, 20.0)
    radf = jnp.floor(ss * 3.0)
    xi = jnp.clip(jnp.floor(sx), 0.0, IMAGE_W - 1.0)
    yi = jnp.clip(jnp.floor(sy), 0.0, IMAGE_H - 1.0)
    big = jnp.float32(1e9)
    lox = jnp.where(valid, xi - radf, big)
    hix = jnp.where(valid, xi + radf, -big)
    loy = jnp.where(valid, yi - radf, big)
    hiy = jnp.where(valid, yi + radf, -big)
    opv = jnp.where(valid, op, 0.0)
    ninv = -0.5 / (ss * ss)

    def put(r, v):
        out[pl.ds(r, 1), :] = v.reshape(1, NG)

    put(0, sx)
    put(1, sy)
    put(2, ninv)
    put(3, opv)
    put(4, lox)
    put(5, hix)
    put(6, loy)
    put(7, hiy)
    put(8, c0)
    put(9, c1)
    put(10, c2)


def _raster_kernel(params, out, cparams, cnt, tbuf):
    @pl.when(pl.program_id(0) == 0)
    def _():
        def cbody(k, c):
            pred = params[7, k] >= params[6, k]

            @pl.when(pred)
            def _():
                for r in range(11):
                    cparams[r, c] = params[r, k]

            return c + jnp.where(pred, 1, 0)

        cnt[0] = jax.lax.fori_loop(0, NG, cbody, jnp.int32(0))

    y0 = pl.program_id(0) * TH
    y0f = jnp.float32(0) + y0
    y1f = y0f + (TH - 1)
    pxf = jax.lax.broadcasted_iota(jnp.int32, (1, IMAGE_W), 1).astype(jnp.float32)
    pyf = y0f + jax.lax.broadcasted_iota(jnp.int32, (TH, 1), 0).astype(jnp.float32)
    out[...] = jnp.zeros((3, TH, IMAGE_W), jnp.float32)
    tbuf[...] = jnp.ones((TH, IMAGE_W), jnp.float32)
    n = cnt[0]

    def body(i, _):
        k = n - 1 - i
        loy = cparams[6, k]
        hiy = cparams[7, k]

        @pl.when((hiy >= y0f) & (loy <= y1f))
        def _():
            sx = cparams[0, k]
            sy = cparams[1, k]
            ninv = cparams[2, k]
            opv = cparams[3, k]
            lox = cparams[4, k]
            hix = cparams[5, k]
            ddx = pxf - sx
            wx = jnp.where((pxf >= lox) & (pxf <= hix),
                           opv * jnp.exp(ninv * (ddx * ddx)), 0.0)
            ddy = pyf - sy
            wy = jnp.where((pyf >= loy) & (pyf <= hiy),
                           jnp.exp(ninv * (ddy * ddy)), 0.0)
            am = (wy * wx) * tbuf[...]
            tbuf[...] = tbuf[...] - am
            out[0] = out[0] + am * cparams[8, k]
            out[1] = out[1] + am * cparams[9, k]
            out[2] = out[2] + am * cparams[10, k]

        return 0

    jax.lax.fori_loop(0, n, body, 0)


def kernel(positions, scales, rotations, colors, opacities, camera_pose):
    del rotations
    R = camera_pose[:3, :3]
    t = camera_pose[:3, 3]
    cam = (positions - t) @ R.T
    gin = jnp.concatenate([cam.T, scales.T, colors.T,
                           opacities[None, :]], axis=0).reshape(10, 8, 256)
    params = pl.pallas_call(
        _project_kernel,
        out_shape=jax.ShapeDtypeStruct((NROWS, NG), jnp.float32),
        in_specs=[pl.BlockSpec((10, 8, 256), lambda: (0, 0, 0))],
        out_specs=pl.BlockSpec((NROWS, NG), lambda: (0, 0)),
    )(gin)

    img = pl.pallas_call(
        _raster_kernel,
        grid=(IMAGE_H // TH,),
        out_shape=jax.ShapeDtypeStruct((3, IMAGE_H, IMAGE_W), jnp.float32),
        in_specs=[pl.BlockSpec((NROWS, NG), lambda i: (0, 0),
                               memory_space=pltpu.SMEM)],
        out_specs=pl.BlockSpec((3, TH, IMAGE_W), lambda i: (0, i, 0)),
        scratch_shapes=[pltpu.SMEM((NROWS, NG), jnp.float32),
                        pltpu.SMEM((1,), jnp.int32),
                        pltpu.VMEM((TH, IMAGE_W), jnp.float32)],
    )(params)
    return img


# speedup vs baseline: 2.4996x; 1.9277x over previous
import jax
import jax.numpy as jnp
import numpy as np
from jax.experimental import pallas as pl
from jax.experimental.pallas import tpu as pltpu

IMAGE_W = 640
IMAGE_H = 480
FOV = 55.0
FOCAL = np.float32(IMAGE_W / (2.0 * np.tan(np.radians(FOV / 2.0))))
NG = 2048
TH = 32
NROWS = 11


def _project_kernel(gin, *outs):
    camx = gin[0]
    camy = gin[1]
    camz = gin[2]
    s0, s1, s2 = gin[3], gin[4], gin[5]
    c0, c1, c2 = gin[6], gin[7], gin[8]
    op = gin[9]
    out, bounds = outs
    depth = jnp.maximum(-camz, 0.1)
    sx = FOCAL * camx / depth + IMAGE_W / 2.0
    sy = FOCAL * camy / depth + IMAGE_H / 2.0
    valid = (sx >= 0) & (sx < IMAGE_W) & (sy >= 0) & (sy < IMAGE_H)
    sm = (s0 + s1 + s2) / 3.0
    ss = jnp.clip(sm * FOCAL / depth, 1.0, 20.0)
    radf = jnp.floor(ss * 3.0)
    xi = jnp.clip(jnp.floor(sx), 0.0, IMAGE_W - 1.0)
    yi = jnp.clip(jnp.floor(sy), 0.0, IMAGE_H - 1.0)
    big = jnp.float32(1e9)
    lox = jnp.where(valid, xi - radf, big)
    hix = jnp.where(valid, xi + radf, -big)
    loy = jnp.where(valid, yi - radf, big)
    hiy = jnp.where(valid, yi + radf, -big)
    opv = jnp.where(valid, op, 0.0)
    ninv = -0.5 / (ss * ss)

    def put(ref, r, v):
        ref[pl.ds(r, 1), :] = v.reshape(1, NG)

    put(bounds, 0, loy)
    put(bounds, 1, hiy)
    put(out, 0, sx)
    put(out, 1, sy)
    put(out, 2, ninv)
    put(out, 3, opv)
    put(out, 4, lox)
    put(out, 5, hix)
    put(out, 6, loy)
    put(out, 7, hiy)
    put(out, 8, c0)
    put(out, 9, c1)
    put(out, 10, c2)


def _raster_kernel(bounds, pvals, out, ids, cnt, tbuf):
    @pl.when(pl.program_id(0) == 0)
    def _():
        def cbody(k, c):
            pred = bounds[1, k] >= bounds[0, k]

            @pl.when(pred)
            def _():
                ids[c] = k

            return c + jnp.where(pred, 1, 0)

        cnt[0] = jax.lax.fori_loop(0, NG, cbody, jnp.int32(0))

    y0 = pl.program_id(0) * TH
    y0f = jnp.float32(0) + y0
    y1f = y0f + (TH - 1)
    pxf = jax.lax.broadcasted_iota(jnp.int32, (1, IMAGE_W), 1).astype(jnp.float32)
    pyf = y0f + jax.lax.broadcasted_iota(jnp.int32, (TH, 1), 0).astype(jnp.float32)
    out[...] = jnp.zeros((3, TH, IMAGE_W), jnp.float32)
    tbuf[...] = jnp.ones((TH, IMAGE_W), jnp.float32)
    n = cnt[0]

    def body(i, _):
        k = ids[n - 1 - i]
        loy_s = bounds[0, k]
        hiy_s = bounds[1, k]

        @pl.when((hiy_s >= y0f) & (loy_s <= y1f))
        def _():
            v = pvals[pl.ds(k, 1), :]
            sx = v[:, 0:1]
            sy = v[:, 1:2]
            ninv = v[:, 2:3]
            opv = v[:, 3:4]
            lox = v[:, 4:5]
            hix = v[:, 5:6]
            loy = v[:, 6:7]
            hiy = v[:, 7:8]
            ddx = pxf - sx
            wx = jnp.where((pxf >= lox) & (pxf <= hix),
                           opv * jnp.exp(ninv * (ddx * ddx)), 0.0)
            ddy = pyf - sy
            wy = jnp.where((pyf >= loy) & (pyf <= hiy),
                           jnp.exp(ninv * (ddy * ddy)), 0.0)
            am = (wy * wx) * tbuf[...]
            tbuf[...] = tbuf[...] - am
            out[0] = out[0] + am * v[:, 8:9]
            out[1] = out[1] + am * v[:, 9:10]
            out[2] = out[2] + am * v[:, 10:11]

        return 0

    jax.lax.fori_loop(0, n, body, 0)


def kernel(positions, scales, rotations, colors, opacities, camera_pose):
    del rotations
    R = camera_pose[:3, :3]
    t = camera_pose[:3, 3]
    cam = (positions - t) @ R.T
    gin = jnp.concatenate([cam.T, scales.T, colors.T,
                           opacities[None, :]], axis=0).reshape(10, 8, 256)
    pvals, bounds = pl.pallas_call(
        _project_kernel,
        out_shape=(jax.ShapeDtypeStruct((NROWS, NG), jnp.float32),
                   jax.ShapeDtypeStruct((2, NG), jnp.float32)),
        in_specs=[pl.BlockSpec((10, 8, 256), lambda: (0, 0, 0))],
        out_specs=(pl.BlockSpec((NROWS, NG), lambda: (0, 0)),
                   pl.BlockSpec((2, NG), lambda: (0, 0))),
    )(gin)

    img = pl.pallas_call(
        _raster_kernel,
        grid=(IMAGE_H // TH,),
        out_shape=jax.ShapeDtypeStruct((3, IMAGE_H, IMAGE_W), jnp.float32),
        in_specs=[pl.BlockSpec((2, NG), lambda i: (0, 0),
                               memory_space=pltpu.SMEM),
                  pl.BlockSpec((NG, NROWS), lambda i: (0, 0))],
        out_specs=pl.BlockSpec((3, TH, IMAGE_W), lambda i: (0, i, 0)),
        scratch_shapes=[pltpu.SMEM((NG,), jnp.int32),
                        pltpu.SMEM((1,), jnp.int32),
                        pltpu.VMEM((TH, IMAGE_W), jnp.float32)],
    )(bounds, pvals.T)
    return img


# speedup vs baseline: 6.6484x; 2.6597x over previous
import jax
import jax.numpy as jnp
import numpy as np
from jax.experimental import pallas as pl
from jax.experimental.pallas import tpu as pltpu

IMAGE_W = 640
IMAGE_H = 480
FOV = 55.0
FOCAL = np.float32(IMAGE_W / (2.0 * np.tan(np.radians(FOV / 2.0))))
NG = 2048
TH = 96
NROWS = 11


def _project_kernel(gin, *outs):
    camx = gin[0]
    camy = gin[1]
    camz = gin[2]
    s0, s1, s2 = gin[3], gin[4], gin[5]
    c0, c1, c2 = gin[6], gin[7], gin[8]
    op = gin[9]
    out, bounds, flags = outs
    depth = jnp.maximum(-camz, 0.1)
    sx = FOCAL * camx / depth + IMAGE_W / 2.0
    sy = FOCAL * camy / depth + IMAGE_H / 2.0
    valid = (sx >= 0) & (sx < IMAGE_W) & (sy >= 0) & (sy < IMAGE_H)
    sm = (s0 + s1 + s2) / 3.0
    ss = jnp.clip(sm * FOCAL / depth, 1.0, 20.0)
    radf = jnp.floor(ss * 3.0)
    xi = jnp.clip(jnp.floor(sx), 0.0, IMAGE_W - 1.0)
    yi = jnp.clip(jnp.floor(sy), 0.0, IMAGE_H - 1.0)
    big = jnp.float32(1e9)
    lox = jnp.where(valid, xi - radf, big)
    hix = jnp.where(valid, xi + radf, -big)
    loy = jnp.where(valid, yi - radf, big)
    hiy = jnp.where(valid, yi + radf, -big)
    opv = jnp.where(valid, op, 0.0)
    ninv = -0.5 / (ss * ss)
    enc = jnp.where(valid, (loy + 60.0) * 1024.0 + (hiy + 60.0), -1.0)
    flags[...] = jnp.max(jnp.where(valid, 1.0, 0.0), axis=1, keepdims=True)

    def put(ref, r, v):
        ref[pl.ds(r, 1), :] = v.reshape(1, NG)

    put(bounds, 0, enc)
    put(out, 0, sx)
    put(out, 1, sy)
    put(out, 2, ninv)
    put(out, 3, opv)
    put(out, 4, lox)
    put(out, 5, hix)
    put(out, 6, loy)
    put(out, 7, hiy)
    put(out, 8, c0)
    put(out, 9, c1)
    put(out, 10, c2)


def _raster_kernel(bounds, flags, pvals, out, ids, cnt, tbuf):
    @pl.when(pl.program_id(0) == 0)
    def _():
        def chunk(r, c):
            def cbody(k, c):
                pred = bounds[0, k] >= 0

                @pl.when(pred)
                def _():
                    ids[c] = k

                return c + jnp.where(pred, 1, 0)

            return jax.lax.cond(
                flags[r, 0] > 0,
                lambda c: jax.lax.fori_loop(r * 256, (r + 1) * 256, cbody, c),
                lambda c: c, c)

        cnt[0] = jax.lax.fori_loop(0, 8, chunk, jnp.int32(0))

    y0 = pl.program_id(0) * TH
    y0f = jnp.float32(0) + y0
    y1f = y0f + (TH - 1)
    pxf = jax.lax.broadcasted_iota(jnp.int32, (1, IMAGE_W), 1).astype(jnp.float32)
    pyf = y0f + jax.lax.broadcasted_iota(jnp.int32, (TH, 1), 0).astype(jnp.float32)
    out[...] = jnp.zeros((3, TH, IMAGE_W), jnp.float32)
    tbuf[...] = jnp.ones((TH, IMAGE_W), jnp.float32)
    n = cnt[0]

    def body(i, _):
        k = ids[n - 1 - i]
        e = bounds[0, k]
        lp = jnp.floor(e * (1.0 / 1024.0))
        hp = e - lp * 1024.0

        @pl.when((hp >= y0f + 60.0) & (lp <= y1f + 60.0))
        def _():
            v = pvals[pl.ds(k, 1), :]
            sx = v[:, 0:1]
            sy = v[:, 1:2]
            ninv = v[:, 2:3]
            opv = v[:, 3:4]
            lox = v[:, 4:5]
            hix = v[:, 5:6]
            loy = v[:, 6:7]
            hiy = v[:, 7:8]
            ddx = pxf - sx
            wx = jnp.where((pxf >= lox) & (pxf <= hix),
                           opv * jnp.exp(ninv * (ddx * ddx)), 0.0)
            ddy = pyf - sy
            wy = jnp.where((pyf >= loy) & (pyf <= hiy),
                           jnp.exp(ninv * (ddy * ddy)), 0.0)
            am = (wy * wx) * tbuf[...]
            tbuf[...] = tbuf[...] - am
            out[0] = out[0] + am * v[:, 8:9]
            out[1] = out[1] + am * v[:, 9:10]
            out[2] = out[2] + am * v[:, 10:11]

        return 0

    jax.lax.fori_loop(0, n, body, 0)


def kernel(positions, scales, rotations, colors, opacities, camera_pose):
    del rotations
    R = camera_pose[:3, :3]
    t = camera_pose[:3, 3]
    cam = (positions - t) @ R.T
    gin = jnp.concatenate([cam.T, scales.T, colors.T,
                           opacities[None, :]], axis=0).reshape(10, 8, 256)
    pvals, bounds, flags = pl.pallas_call(
        _project_kernel,
        out_shape=(jax.ShapeDtypeStruct((NROWS, NG), jnp.float32),
                   jax.ShapeDtypeStruct((1, NG), jnp.float32),
                   jax.ShapeDtypeStruct((8, 1), jnp.float32)),
        in_specs=[pl.BlockSpec((10, 8, 256), lambda: (0, 0, 0))],
        out_specs=(pl.BlockSpec((NROWS, NG), lambda: (0, 0)),
                   pl.BlockSpec((1, NG), lambda: (0, 0)),
                   pl.BlockSpec((8, 1), lambda: (0, 0))),
    )(gin)

    img = pl.pallas_call(
        _raster_kernel,
        grid=(IMAGE_H // TH,),
        out_shape=jax.ShapeDtypeStruct((3, IMAGE_H, IMAGE_W), jnp.float32),
        in_specs=[pl.BlockSpec((1, NG), lambda i: (0, 0),
                               memory_space=pltpu.SMEM),
                  pl.BlockSpec((8, 1), lambda i: (0, 0),
                               memory_space=pltpu.SMEM),
                  pl.BlockSpec((NG, NROWS), lambda i: (0, 0))],
        out_specs=pl.BlockSpec((3, TH, IMAGE_W), lambda i: (0, i, 0)),
        scratch_shapes=[pltpu.SMEM((NG,), jnp.int32),
                        pltpu.SMEM((1,), jnp.int32),
                        pltpu.VMEM((TH, IMAGE_W), jnp.float32)],
    )(bounds, flags, pvals.T)
    return img


# speedup vs baseline: 6.7262x; 1.0117x over previous
import jax
import jax.numpy as jnp
import numpy as np
from jax.experimental import pallas as pl
from jax.experimental.pallas import tpu as pltpu

IMAGE_W = 640
IMAGE_H = 480
FOV = 55.0
FOCAL = np.float32(IMAGE_W / (2.0 * np.tan(np.radians(FOV / 2.0))))
NG = 2048
TH = 480
NROWS = 11


def _project_kernel(gin, *outs):
    camx = gin[0]
    camy = gin[1]
    camz = gin[2]
    s0, s1, s2 = gin[3], gin[4], gin[5]
    c0, c1, c2 = gin[6], gin[7], gin[8]
    op = gin[9]
    out, bounds, flags = outs
    depth = jnp.maximum(-camz, 0.1)
    sx = FOCAL * camx / depth + IMAGE_W / 2.0
    sy = FOCAL * camy / depth + IMAGE_H / 2.0
    valid = (sx >= 0) & (sx < IMAGE_W) & (sy >= 0) & (sy < IMAGE_H)
    sm = (s0 + s1 + s2) / 3.0
    ss = jnp.clip(sm * FOCAL / depth, 1.0, 20.0)
    radf = jnp.floor(ss * 3.0)
    xi = jnp.clip(jnp.floor(sx), 0.0, IMAGE_W - 1.0)
    yi = jnp.clip(jnp.floor(sy), 0.0, IMAGE_H - 1.0)
    big = jnp.float32(1e9)
    lox = jnp.where(valid, xi - radf, big)
    hix = jnp.where(valid, xi + radf, -big)
    loy = jnp.where(valid, yi - radf, big)
    hiy = jnp.where(valid, yi + radf, -big)
    opv = jnp.where(valid, op, 0.0)
    ninv = -0.5 / (ss * ss)
    enc = jnp.where(valid, (loy + 60.0) * 1024.0 + (hiy + 60.0), -1.0)
    flags[...] = jnp.max(jnp.where(valid, 1.0, 0.0), axis=1, keepdims=True)

    def put(ref, r, v):
        ref[pl.ds(r, 1), :] = v.reshape(1, NG)

    put(bounds, 0, enc)
    put(out, 0, sx)
    put(out, 1, sy)
    put(out, 2, ninv)
    put(out, 3, opv)
    put(out, 4, lox)
    put(out, 5, hix)
    put(out, 6, loy)
    put(out, 7, hiy)
    put(out, 8, c0)
    put(out, 9, c1)
    put(out, 10, c2)


def _raster_kernel(bounds, flags, pvals, out, ids, cnt, tbuf):
    @pl.when(pl.program_id(0) == 0)
    def _():
        def chunk(r, c):
            def cbody(k, c):
                pred = bounds[0, k] >= 0

                @pl.when(pred)
                def _():
                    ids[c] = k

                return c + jnp.where(pred, 1, 0)

            return jax.lax.cond(
                flags[r, 0] > 0,
                lambda c: jax.lax.fori_loop(r * 256, (r + 1) * 256, cbody, c),
                lambda c: c, c)

        cnt[0] = jax.lax.fori_loop(0, 8, chunk, jnp.int32(0))

    y0 = pl.program_id(0) * TH
    y0f = jnp.float32(0) + y0
    y1f = y0f + (TH - 1)
    pxf = jax.lax.broadcasted_iota(jnp.int32, (1, IMAGE_W), 1).astype(jnp.float32)
    pyf = y0f + jax.lax.broadcasted_iota(jnp.int32, (TH, 1), 0).astype(jnp.float32)
    out[...] = jnp.zeros((3, TH, IMAGE_W), jnp.float32)
    tbuf[...] = jnp.ones((TH, IMAGE_W), jnp.float32)
    n = cnt[0]

    def body(i, _):
        k = ids[n - 1 - i]
        e = bounds[0, k]
        lp = jnp.floor(e * (1.0 / 1024.0))
        hp = e - lp * 1024.0

        @pl.when((hp >= y0f + 60.0) & (lp <= y1f + 60.0))
        def _():
            v = pvals[pl.ds(k, 1), :]
            sx = v[:, 0:1]
            sy = v[:, 1:2]
            ninv = v[:, 2:3]
            opv = v[:, 3:4]
            lox = v[:, 4:5]
            hix = v[:, 5:6]
            loy = v[:, 6:7]
            hiy = v[:, 7:8]
            ddx = pxf - sx
            wx = jnp.where((pxf >= lox) & (pxf <= hix),
                           opv * jnp.exp(ninv * (ddx * ddx)), 0.0)
            ddy = pyf - sy
            wy = jnp.where((pyf >= loy) & (pyf <= hiy),
                           jnp.exp(ninv * (ddy * ddy)), 0.0)
            am = (wy * wx) * tbuf[...]
            tbuf[...] = tbuf[...] - am
            out[0] = out[0] + am * v[:, 8:9]
            out[1] = out[1] + am * v[:, 9:10]
            out[2] = out[2] + am * v[:, 10:11]

        return 0

    jax.lax.fori_loop(0, n, body, 0)


def kernel(positions, scales, rotations, colors, opacities, camera_pose):
    del rotations
    R = camera_pose[:3, :3]
    t = camera_pose[:3, 3]
    cam = (positions - t) @ R.T
    gin = jnp.concatenate([cam.T, scales.T, colors.T,
                           opacities[None, :]], axis=0).reshape(10, 8, 256)
    pvals, bounds, flags = pl.pallas_call(
        _project_kernel,
        out_shape=(jax.ShapeDtypeStruct((NROWS, NG), jnp.float32),
                   jax.ShapeDtypeStruct((1, NG), jnp.float32),
                   jax.ShapeDtypeStruct((8, 1), jnp.float32)),
        in_specs=[pl.BlockSpec((10, 8, 256), lambda: (0, 0, 0))],
        out_specs=(pl.BlockSpec((NROWS, NG), lambda: (0, 0)),
                   pl.BlockSpec((1, NG), lambda: (0, 0)),
                   pl.BlockSpec((8, 1), lambda: (0, 0))),
    )(gin)

    img = pl.pallas_call(
        _raster_kernel,
        grid=(IMAGE_H // TH,),
        out_shape=jax.ShapeDtypeStruct((3, IMAGE_H, IMAGE_W), jnp.float32),
        in_specs=[pl.BlockSpec((1, NG), lambda i: (0, 0),
                               memory_space=pltpu.SMEM),
                  pl.BlockSpec((8, 1), lambda i: (0, 0),
                               memory_space=pltpu.SMEM),
                  pl.BlockSpec((NG, NROWS), lambda i: (0, 0))],
        out_specs=pl.BlockSpec((3, TH, IMAGE_W), lambda i: (0, i, 0)),
        scratch_shapes=[pltpu.SMEM((NG,), jnp.int32),
                        pltpu.SMEM((1,), jnp.int32),
                        pltpu.VMEM((TH, IMAGE_W), jnp.float32)],
    )(bounds, flags, pvals.T)
    return img


# speedup vs baseline: 7.4628x; 1.1095x over previous
import jax
import jax.numpy as jnp
import numpy as np
from jax.experimental import pallas as pl
from jax.experimental.pallas import tpu as pltpu

IMAGE_W = 640
IMAGE_H = 480
FOV = 55.0
FOCAL = np.float32(IMAGE_W / (2.0 * np.tan(np.radians(FOV / 2.0))))
NG = 2048
TH = 96
NROWS = 11


def _project_kernel(gin, *outs):
    camx = gin[0]
    camy = gin[1]
    camz = gin[2]
    s0, s1, s2 = gin[3], gin[4], gin[5]
    c0, c1, c2 = gin[6], gin[7], gin[8]
    op = gin[9]
    out, bounds, flags = outs
    depth = jnp.maximum(-camz, 0.1)
    sx = FOCAL * camx / depth + IMAGE_W / 2.0
    sy = FOCAL * camy / depth + IMAGE_H / 2.0
    valid = (sx >= 0) & (sx < IMAGE_W) & (sy >= 0) & (sy < IMAGE_H)
    sm = (s0 + s1 + s2) / 3.0
    ss = jnp.clip(sm * FOCAL / depth, 1.0, 20.0)
    radf = jnp.floor(ss * 3.0)
    xi = jnp.clip(jnp.floor(sx), 0.0, IMAGE_W - 1.0)
    yi = jnp.clip(jnp.floor(sy), 0.0, IMAGE_H - 1.0)
    big = jnp.float32(1e9)
    lox = jnp.where(valid, xi - radf, big)
    hix = jnp.where(valid, xi + radf, -big)
    loy = jnp.where(valid, yi - radf, big)
    hiy = jnp.where(valid, yi + radf, -big)
    opv = jnp.where(valid, op, 0.0)
    ninv = -0.5 / (ss * ss)
    enc = jnp.where(valid, (loy + 60.0) * 1024.0 + (hiy + 60.0), -1.0)
    flags[...] = jnp.max(jnp.where(valid, 1.0, 0.0), axis=1, keepdims=True)

    bounds[pl.ds(0, 1), :] = enc.reshape(1, NG)
    tab = jnp.concatenate(
        [v.reshape(1, NG) for v in
         (sx, sy, ninv, opv, lox, hix, loy, hiy, c0, c1, c2)], axis=0)
    out[...] = tab.T


def _raster_kernel(bounds, flags, pvals, out, ids, cnt, tbuf):
    @pl.when(pl.program_id(0) == 0)
    def _():
        def chunk(r, c):
            def cbody(k, c):
                pred = bounds[0, k] >= 0

                @pl.when(pred)
                def _():
                    ids[c] = k

                return c + jnp.where(pred, 1, 0)

            return jax.lax.cond(
                flags[r, 0] > 0,
                lambda c: jax.lax.fori_loop(r * 256, (r + 1) * 256, cbody, c),
                lambda c: c, c)

        cnt[0] = jax.lax.fori_loop(0, 8, chunk, jnp.int32(0))

    y0 = pl.program_id(0) * TH
    y0f = jnp.float32(0) + y0
    y1f = y0f + (TH - 1)
    pxf = jax.lax.broadcasted_iota(jnp.int32, (1, IMAGE_W), 1).astype(jnp.float32)
    pyf = y0f + jax.lax.broadcasted_iota(jnp.int32, (TH, 1), 0).astype(jnp.float32)
    out[...] = jnp.zeros((3, TH, IMAGE_W), jnp.float32)
    tbuf[...] = jnp.ones((TH, IMAGE_W), jnp.float32)
    n = cnt[0]

    def body(i, _):
        k = ids[n - 1 - i]
        e = bounds[0, k]
        lp = jnp.floor(e * (1.0 / 1024.0))
        hp = e - lp * 1024.0

        @pl.when((hp >= y0f + 60.0) & (lp <= y1f + 60.0))
        def _():
            v = pvals[pl.ds(k, 1), :]
            sx = v[:, 0:1]
            sy = v[:, 1:2]
            ninv = v[:, 2:3]
            opv = v[:, 3:4]
            lox = v[:, 4:5]
            hix = v[:, 5:6]
            loy = v[:, 6:7]
            hiy = v[:, 7:8]
            ddx = pxf - sx
            wx = jnp.where((pxf >= lox) & (pxf <= hix),
                           opv * jnp.exp(ninv * (ddx * ddx)), 0.0)
            ddy = pyf - sy
            wy = jnp.where((pyf >= loy) & (pyf <= hiy),
                           jnp.exp(ninv * (ddy * ddy)), 0.0)
            am = (wy * wx) * tbuf[...]
            tbuf[...] = tbuf[...] - am
            out[0] = out[0] + am * v[:, 8:9]
            out[1] = out[1] + am * v[:, 9:10]
            out[2] = out[2] + am * v[:, 10:11]

        return 0

    jax.lax.fori_loop(0, n, body, 0)


def kernel(positions, scales, rotations, colors, opacities, camera_pose):
    del rotations
    R = camera_pose[:3, :3]
    t = camera_pose[:3, 3]
    cam = (positions - t) @ R.T
    gin = jnp.concatenate([cam.T, scales.T, colors.T,
                           opacities[None, :]], axis=0).reshape(10, 8, 256)
    pvals, bounds, flags = pl.pallas_call(
        _project_kernel,
        out_shape=(jax.ShapeDtypeStruct((NG, NROWS), jnp.float32),
                   jax.ShapeDtypeStruct((1, NG), jnp.float32),
                   jax.ShapeDtypeStruct((8, 1), jnp.float32)),
        in_specs=[pl.BlockSpec((10, 8, 256), lambda: (0, 0, 0))],
        out_specs=(pl.BlockSpec((NG, NROWS), lambda: (0, 0)),
                   pl.BlockSpec((1, NG), lambda: (0, 0)),
                   pl.BlockSpec((8, 1), lambda: (0, 0))),
    )(gin)

    img = pl.pallas_call(
        _raster_kernel,
        grid=(IMAGE_H // TH,),
        out_shape=jax.ShapeDtypeStruct((3, IMAGE_H, IMAGE_W), jnp.float32),
        in_specs=[pl.BlockSpec((1, NG), lambda i: (0, 0),
                               memory_space=pltpu.SMEM),
                  pl.BlockSpec((8, 1), lambda i: (0, 0),
                               memory_space=pltpu.SMEM),
                  pl.BlockSpec((NG, NROWS), lambda i: (0, 0))],
        out_specs=pl.BlockSpec((3, TH, IMAGE_W), lambda i: (0, i, 0)),
        scratch_shapes=[pltpu.SMEM((NG,), jnp.int32),
                        pltpu.SMEM((1,), jnp.int32),
                        pltpu.VMEM((TH, IMAGE_W), jnp.float32)],
    )(bounds, flags, pvals)
    return img


# speedup vs baseline: 8.5082x; 1.1401x over previous
import jax
import jax.numpy as jnp
import numpy as np
from jax.experimental import pallas as pl
from jax.experimental.pallas import tpu as pltpu

IMAGE_W = 640
IMAGE_H = 480
FOV = 55.0
FOCAL = np.float32(IMAGE_W / (2.0 * np.tan(np.radians(FOV / 2.0))))
NG = 2048
TH = 96
NROWS = 11


def _render_kernel(gin, out, pvals, ids, cnt, sbounds, sflags, tbuf,
                   vbounds, vflags, sem0, sem1):
    @pl.when(pl.program_id(0) == 0)
    def _():
        camx = gin[0]
        camy = gin[1]
        camz = gin[2]
        s0, s1, s2 = gin[3], gin[4], gin[5]
        c0, c1, c2 = gin[6], gin[7], gin[8]
        op = gin[9]
        depth = jnp.maximum(-camz, 0.1)
        sx = FOCAL * camx / depth + IMAGE_W / 2.0
        sy = FOCAL * camy / depth + IMAGE_H / 2.0
        valid = (sx >= 0) & (sx < IMAGE_W) & (sy >= 0) & (sy < IMAGE_H)
        sm = (s0 + s1 + s2) / 3.0
        ss = jnp.clip(sm * FOCAL / depth, 1.0, 20.0)
        radf = jnp.floor(ss * 3.0)
        xi = jnp.clip(jnp.floor(sx), 0.0, IMAGE_W - 1.0)
        yi = jnp.clip(jnp.floor(sy), 0.0, IMAGE_H - 1.0)
        big = jnp.float32(1e9)
        lox = jnp.where(valid, xi - radf, big)
        hix = jnp.where(valid, xi + radf, -big)
        loy = jnp.where(valid, yi - radf, big)
        hiy = jnp.where(valid, yi + radf, -big)
        opv = jnp.where(valid, op, 0.0)
        ninv = -0.5 / (ss * ss)
        enc = jnp.where(valid, (loy + 60.0) * 1024.0 + (hiy + 60.0), -1.0)
        vbounds[...] = enc.reshape(1, NG)
        vflags[...] = jnp.max(jnp.where(valid, 1.0, 0.0), axis=1,
                              keepdims=True)
        tab = jnp.concatenate(
            [v.reshape(1, NG) for v in
             (sx, sy, ninv, opv, lox, hix, loy, hiy, c0, c1, c2)], axis=0)
        pvals[...] = tab.T

        cp0 = pltpu.make_async_copy(vbounds, sbounds, sem0)
        cp1 = pltpu.make_async_copy(vflags, sflags, sem1)
        cp0.start()
        cp1.start()
        cp0.wait()
        cp1.wait()

        def chunk(r, c):
            def cbody(k, c):
                pred = sbounds[0, k] >= 0

                @pl.when(pred)
                def _():
                    ids[c] = k

                return c + jnp.where(pred, 1, 0)

            return jax.lax.cond(
                sflags[r, 0] > 0,
                lambda c: jax.lax.fori_loop(r * 256, (r + 1) * 256, cbody, c),
                lambda c: c, c)

        cnt[0] = jax.lax.fori_loop(0, 8, chunk, jnp.int32(0))

    y0 = pl.program_id(0) * TH
    y0f = jnp.float32(0) + y0
    y1f = y0f + (TH - 1)
    pxf = jax.lax.broadcasted_iota(jnp.int32, (1, IMAGE_W), 1).astype(jnp.float32)
    pyf = y0f + jax.lax.broadcasted_iota(jnp.int32, (TH, 1), 0).astype(jnp.float32)
    out[...] = jnp.zeros((3, TH, IMAGE_W), jnp.float32)
    tbuf[...] = jnp.ones((TH, IMAGE_W), jnp.float32)
    n = cnt[0]

    def body(i, _):
        k = ids[n - 1 - i]
        e = sbounds[0, k]
        lp = jnp.floor(e * (1.0 / 1024.0))
        hp = e - lp * 1024.0

        @pl.when((hp >= y0f + 60.0) & (lp <= y1f + 60.0))
        def _():
            v = pvals[pl.ds(k, 1), :]
            sx = v[:, 0:1]
            sy = v[:, 1:2]
            ninv = v[:, 2:3]
            opv = v[:, 3:4]
            lox = v[:, 4:5]
            hix = v[:, 5:6]
            loy = v[:, 6:7]
            hiy = v[:, 7:8]
            ddx = pxf - sx
            wx = jnp.where((pxf >= lox) & (pxf <= hix),
                           opv * jnp.exp(ninv * (ddx * ddx)), 0.0)
            ddy = pyf - sy
            wy = jnp.where((pyf >= loy) & (pyf <= hiy),
                           jnp.exp(ninv * (ddy * ddy)), 0.0)
            am = (wy * wx) * tbuf[...]
            tbuf[...] = tbuf[...] - am
            out[0] = out[0] + am * v[:, 8:9]
            out[1] = out[1] + am * v[:, 9:10]
            out[2] = out[2] + am * v[:, 10:11]

        return 0

    jax.lax.fori_loop(0, n, body, 0)


def kernel(positions, scales, rotations, colors, opacities, camera_pose):
    del rotations
    R = camera_pose[:3, :3]
    t = camera_pose[:3, 3]
    cam = (positions - t) @ R.T
    gin = jnp.concatenate([cam.T, scales.T, colors.T,
                           opacities[None, :]], axis=0).reshape(10, 8, 256)
    img = pl.pallas_call(
        _render_kernel,
        grid=(IMAGE_H // TH,),
        out_shape=jax.ShapeDtypeStruct((3, IMAGE_H, IMAGE_W), jnp.float32),
        in_specs=[pl.BlockSpec((10, 8, 256), lambda i: (0, 0, 0))],
        out_specs=pl.BlockSpec((3, TH, IMAGE_W), lambda i: (0, i, 0)),
        scratch_shapes=[pltpu.VMEM((NG, NROWS), jnp.float32),
                        pltpu.SMEM((NG,), jnp.int32),
                        pltpu.SMEM((1,), jnp.int32),
                        pltpu.SMEM((1, NG), jnp.float32),
                        pltpu.SMEM((8, 1), jnp.float32),
                        pltpu.VMEM((TH, IMAGE_W), jnp.float32),
                        pltpu.VMEM((1, NG), jnp.float32),
                        pltpu.VMEM((8, 1), jnp.float32),
                        pltpu.SemaphoreType.DMA,
                        pltpu.SemaphoreType.DMA],
    )(gin)
    return img
